# baseline (speedup 1.0000x reference)
import functools

import jax
import jax.numpy as jnp
from jax import lax
from jax.experimental import pallas as pl
from jax.experimental.pallas import tpu as pltpu
from jax.experimental.pallas import tpu_sc as plsc

E = 64
NC, NS, L = 2, 16, 16
NW = NC * NS
IDXW = 128
K = 4
CHUNK = K * IDXW


def _sc_gather(table, idx3d, n_rows):
    rows_per_w = n_rows // NW
    idx_rows = rows_per_w // IDXW
    n_chunks = rows_per_w // CHUNK

    mesh = plsc.VectorSubcoreMesh(core_axis_name="c", subcore_axis_name="s")

    @functools.partial(
        pl.kernel,
        mesh=mesh,
        out_type=jax.ShapeDtypeStruct((n_rows, E), jnp.float32),
        scratch_types=[
            pltpu.VMEM((idx_rows, IDXW), jnp.int32),
            pltpu.VMEM((CHUNK, E), jnp.float32),
            pltpu.SemaphoreType.DMA,
        ],
        compiler_params=pltpu.CompilerParams(use_tc_tiling_on_sc=False),
    )
    def gather_kernel(table_hbm, idx_hbm, out_hbm, idx_v, rows_v, sem):
        wid = lax.axis_index("s") * NC + lax.axis_index("c")
        w_base = wid * rows_per_w
        pltpu.sync_copy(idx_hbm.at[wid], idx_v)

        def body(g, carry):
            base = w_base + g * CHUNK
            copies = []
            for j in range(K):
                copies.append(
                    pltpu.async_copy(
                        table_hbm.at[idx_v.at[g * K + j]],
                        rows_v.at[pl.ds(j * IDXW, IDXW)],
                        sem,
                    )
                )
            for c in copies:
                c.wait()
            pltpu.sync_copy(rows_v, out_hbm.at[pl.ds(base, CHUNK)])
            return carry

        lax.fori_loop(0, n_chunks, body, 0)

    return gather_kernel(table, idx3d)


def _mm_body(x_ref, w_ref, o_ref):
    o_ref[0] = jnp.dot(x_ref[0], w_ref[0], preferred_element_type=jnp.float32)


def _tc_matmul(x, wt, blk):
    n = x.shape[1]
    grid = (2, n // blk)
    return pl.pallas_call(
        _mm_body,
        grid=grid,
        in_specs=[
            pl.BlockSpec((1, blk, E), lambda s, j: (s, j, 0)),
            pl.BlockSpec((1, E, E), lambda s, j: (s, 0, 0)),
        ],
        out_specs=pl.BlockSpec((1, blk, E), lambda s, j: (s, j, 0)),
        out_shape=jax.ShapeDtypeStruct((2, n, E), jnp.float32),
    )(x, wt)


def kernel(sent1, sent2, table, W1, W2):
    b, sl = sent1.shape
    n_half = b * sl
    n_rows = 2 * n_half
    idx = jnp.concatenate(
        [sent1.reshape(-1), sent2.reshape(-1)]
    ).astype(jnp.int32)
    idx3d = idx.reshape(NW, -1, IDXW)

    e = _sc_gather(table, idx3d, n_rows)

    wt = jnp.stack([W1.T, W2.T])
    out = _tc_matmul(e.reshape(2, n_half, E), wt, blk=4096)

    s1 = out[0].reshape(b, sl, E)
    s2 = out[1].reshape(b, sl, E)
    return (s1, s2)


# speedup vs baseline: 1.1080x; 1.1080x over previous
import functools

import jax
import jax.numpy as jnp
from jax import lax
from jax.experimental import pallas as pl
from jax.experimental.pallas import tpu as pltpu
from jax.experimental.pallas import tpu_sc as plsc

E = 64
NC, NS, L = 2, 16, 16
NW = NC * NS
IDXW = 128
K = 4
CHUNK = K * IDXW


def _sc_gather(table, idx3d, n_rows):
    rows_per_w = n_rows // NW
    idx_rows = rows_per_w // IDXW
    n_chunks = rows_per_w // CHUNK

    mesh = plsc.VectorSubcoreMesh(core_axis_name="c", subcore_axis_name="s")

    @functools.partial(
        pl.kernel,
        mesh=mesh,
        out_type=jax.ShapeDtypeStruct((n_rows, E), jnp.float32),
        scratch_types=[
            pltpu.VMEM((idx_rows, IDXW), jnp.int32),
            pltpu.VMEM((CHUNK, E), jnp.float32),
            pltpu.SemaphoreType.DMA,
        ],
        compiler_params=pltpu.CompilerParams(use_tc_tiling_on_sc=False),
    )
    def gather_kernel(table_hbm, idx_hbm, out_hbm, idx_v, rows_v, sem):
        wid = lax.axis_index("s") * NC + lax.axis_index("c")
        w_base = wid * rows_per_w
        pltpu.sync_copy(idx_hbm.at[wid], idx_v)

        def body(g, carry):
            base = w_base + g * CHUNK
            copies = []
            for j in range(K):
                copies.append(
                    pltpu.async_copy(
                        table_hbm.at[idx_v.at[g * K + j]],
                        rows_v.at[pl.ds(j * IDXW, IDXW)],
                        sem,
                    )
                )
            for c in copies:
                c.wait()
            pltpu.sync_copy(rows_v, out_hbm.at[pl.ds(base, CHUNK)])
            return carry

        lax.fori_loop(0, n_chunks, body, 0)

    return gather_kernel(table, idx3d)


def _mm_body(x_ref, w_ref, o_ref):
    o_ref[0] = lax.dot_general(
        w_ref[...],
        x_ref[0],
        dimension_numbers=(((1,), (1,)), ((), ())),
        preferred_element_type=jnp.float32,
    )


def _tc_matmul(x, w, sl, b):
    return pl.pallas_call(
        _mm_body,
        grid=(sl,),
        in_specs=[
            pl.BlockSpec((1, b, E), lambda l: (l, 0, 0)),
            pl.BlockSpec((E, E), lambda l: (0, 0)),
        ],
        out_specs=pl.BlockSpec((1, E, b), lambda l: (l, 0, 0)),
        out_shape=jax.ShapeDtypeStruct((sl, E, b), jnp.float32),
    )(x, w)


def kernel(sent1, sent2, table, W1, W2):
    b, sl = sent1.shape
    n_rows = 2 * b * sl
    idx = jnp.concatenate(
        [sent1.T.reshape(-1), sent2.T.reshape(-1)]
    ).astype(jnp.int32)
    idx3d = idx.reshape(NW, -1, IDXW)

    e = _sc_gather(table, idx3d, n_rows)
    e3 = e.reshape(2 * sl, b, E)

    o1 = _tc_matmul(e3[:sl], W1, sl, b)
    o2 = _tc_matmul(e3[sl:], W2, sl, b)

    s1 = o1.transpose(2, 0, 1)
    s2 = o2.transpose(2, 0, 1)
    return (s1, s2)


# speedup vs baseline: 2.3606x; 2.1305x over previous
import functools

import jax
import jax.numpy as jnp
from jax import lax
from jax.experimental import pallas as pl
from jax.experimental.pallas import tpu as pltpu
from jax.experimental.pallas import tpu_sc as plsc

E = 64
NC, NS, L = 2, 16, 16
NW = NC * NS
IDXW = 128
K = 4
CHUNK = K * IDXW


def _sc_gather(table, idx3d, n_rows):
    rows_per_w = n_rows // NW
    idx_rows = rows_per_w // IDXW
    n_chunks = rows_per_w // CHUNK

    mesh = plsc.VectorSubcoreMesh(core_axis_name="c", subcore_axis_name="s")

    @functools.partial(
        pl.kernel,
        mesh=mesh,
        out_type=jax.ShapeDtypeStruct((n_rows, E), jnp.float32),
        scratch_types=[
            pltpu.VMEM((idx_rows, IDXW), jnp.int32),
            pltpu.VMEM((CHUNK, E), jnp.float32),
            pltpu.SemaphoreType.DMA,
        ],
        compiler_params=pltpu.CompilerParams(use_tc_tiling_on_sc=False),
    )
    def gather_kernel(table_hbm, idx_hbm, out_hbm, idx_v, rows_v, sem):
        wid = lax.axis_index("s") * NC + lax.axis_index("c")
        w_base = wid * rows_per_w
        pltpu.sync_copy(idx_hbm.at[wid], idx_v)

        def body(g, carry):
            base = w_base + g * CHUNK
            copies = []
            for j in range(K):
                copies.append(
                    pltpu.async_copy(
                        table_hbm.at[idx_v.at[g * K + j]],
                        rows_v.at[pl.ds(j * IDXW, IDXW)],
                        sem,
                    )
                )
            for c in copies:
                c.wait()
            pltpu.sync_copy(rows_v, out_hbm.at[pl.ds(base, CHUNK)])
            return carry

        lax.fori_loop(0, n_chunks, body, 0)

    return gather_kernel(table, idx3d)


def _pack_body(xa_ref, xb_ref, o_ref):
    o_ref[:, 0:E] = xa_ref[...].T
    o_ref[:, E:2 * E] = xb_ref[...].T


def _tc_pack(tt, n):
    BP = 4096
    g = -(-n // (2 * BP))
    nblk = -(-n // BP) - 1
    return pl.pallas_call(
        _pack_body,
        grid=(g,),
        in_specs=[
            pl.BlockSpec((E, BP), lambda i: (0, jnp.minimum(2 * i, nblk))),
            pl.BlockSpec(
                (E, BP), lambda i: (0, jnp.minimum(2 * i + 1, nblk))
            ),
        ],
        out_specs=pl.BlockSpec((BP, 2 * E), lambda i: (i, 0)),
        out_shape=jax.ShapeDtypeStruct((g * BP, 2 * E), jnp.float32),
    )(tt, tt)


def _mm_body(x_ref, w_ref, o_ref):
    hb = x_ref.shape[1]
    dn = (((1,), (1,)), ((), ()))
    o_ref[0, :, 0:hb] = lax.dot_general(
        w_ref[...], x_ref[0, :, 0:E], dn, preferred_element_type=jnp.float32
    )
    o_ref[0, :, hb:2 * hb] = lax.dot_general(
        w_ref[...], x_ref[0, :, E:2 * E], dn,
        preferred_element_type=jnp.float32,
    )


def _tc_matmul(x2, w, sl, b, off):
    hb = b // 2
    return pl.pallas_call(
        _mm_body,
        grid=(sl,),
        in_specs=[
            pl.BlockSpec((1, hb, 2 * E), lambda l: (l + off, 0, 0)),
            pl.BlockSpec((E, E), lambda l: (0, 0)),
        ],
        out_specs=pl.BlockSpec((1, E, b), lambda l: (l, 0, 0)),
        out_shape=jax.ShapeDtypeStruct((sl, E, b), jnp.float32),
    )(x2, w)


def kernel(sent1, sent2, table, W1, W2):
    b, sl = sent1.shape
    n_rows = 2 * b * sl
    idx = jnp.concatenate(
        [sent1.T.reshape(-1), sent2.T.reshape(-1)]
    ).astype(jnp.int32)
    jdx = (
        ((idx >> 13) << 13) + 2 * (idx & 4095) + ((idx >> 12) & 1)
    )
    jdx = (
        jdx.reshape(2 * sl, 2, b // 2).transpose(0, 2, 1).reshape(-1)
    )
    idx3d = jdx.reshape(NW, -1, IDXW)

    n = table.shape[0]
    packed = _tc_pack(table.T, n)
    src = packed.reshape(-1, E)

    e = _sc_gather(src, idx3d, n_rows)
    e3 = e.reshape(2 * sl, b // 2, 2 * E)

    o1 = _tc_matmul(e3, W1, sl, b, 0)
    o2 = _tc_matmul(e3, W2, sl, b, sl)

    s1 = o1.transpose(2, 0, 1)
    s2 = o2.transpose(2, 0, 1)
    return (s1, s2)


# speedup vs baseline: 2.9302x; 1.2413x over previous
import functools

import jax
import jax.numpy as jnp
from jax import lax
from jax.experimental import pallas as pl
from jax.experimental.pallas import tpu as pltpu
from jax.experimental.pallas import tpu_sc as plsc

E = 64
NC, NS, L = 2, 16, 16
NW = NC * NS
IDXW = 128
K = 10
CHUNK = K * IDXW


def _sc_gather(table, idx3d, n_rows):
    rows_per_w = n_rows // NW
    idx_rows = rows_per_w // IDXW
    n_chunks = rows_per_w // CHUNK

    mesh = plsc.VectorSubcoreMesh(core_axis_name="c", subcore_axis_name="s")

    @functools.partial(
        pl.kernel,
        mesh=mesh,
        out_type=jax.ShapeDtypeStruct((n_rows, E), jnp.float32),
        scratch_types=[
            pltpu.VMEM((idx_rows, IDXW), jnp.int32),
            pltpu.VMEM((CHUNK, E), jnp.float32),
            pltpu.SemaphoreType.DMA,
        ],
        compiler_params=pltpu.CompilerParams(use_tc_tiling_on_sc=False),
    )
    def gather_kernel(table_hbm, idx_hbm, out_hbm, idx_v, rows_v, sem):
        wid = lax.axis_index("s") * NC + lax.axis_index("c")
        w_base = wid * rows_per_w
        pltpu.sync_copy(idx_hbm.at[wid], idx_v)

        def body(g, carry):
            base = w_base + g * CHUNK
            copies = []
            for j in range(K):
                copies.append(
                    pltpu.async_copy(
                        table_hbm.at[idx_v.at[g * K + j]],
                        rows_v.at[pl.ds(j * IDXW, IDXW)],
                        sem,
                    )
                )
            for c in copies:
                c.wait()
            pltpu.sync_copy(rows_v, out_hbm.at[pl.ds(base, CHUNK)])
            return carry

        lax.fori_loop(0, n_chunks, body, 0)

    return gather_kernel(table, idx3d)


def _pack_body(xa_ref, xb_ref, o_ref):
    o_ref[:, 0:E] = xa_ref[...].T
    o_ref[:, E:2 * E] = xb_ref[...].T


BP = 16384


def _tc_pack(tt, n):
    g = -(-n // (2 * BP))
    nblk = -(-n // BP) - 1
    return pl.pallas_call(
        _pack_body,
        grid=(g,),
        in_specs=[
            pl.BlockSpec((E, BP), lambda i: (0, jnp.minimum(2 * i, nblk))),
            pl.BlockSpec(
                (E, BP), lambda i: (0, jnp.minimum(2 * i + 1, nblk))
            ),
        ],
        out_specs=pl.BlockSpec((BP, 2 * E), lambda i: (i, 0)),
        out_shape=jax.ShapeDtypeStruct((g * BP, 2 * E), jnp.float32),
    )(tt, tt)


LB = 5


def _mm_body(x_ref, w_ref, o_ref):
    hb = x_ref.shape[1]
    dn = (((1,), (1,)), ((), ()))
    for r in range(LB):
        o_ref[r, :, 0:hb] = lax.dot_general(
            w_ref[...], x_ref[r, :, 0:E], dn,
            preferred_element_type=jnp.float32,
        )
        o_ref[r, :, hb:2 * hb] = lax.dot_general(
            w_ref[...], x_ref[r, :, E:2 * E], dn,
            preferred_element_type=jnp.float32,
        )


def _tc_matmul(x2, w, sl, b, off):
    hb = b // 2
    return pl.pallas_call(
        _mm_body,
        grid=(sl // LB,),
        in_specs=[
            pl.BlockSpec((LB, hb, 2 * E), lambda l: (l + off, 0, 0)),
            pl.BlockSpec((E, E), lambda l: (0, 0)),
        ],
        out_specs=pl.BlockSpec((LB, E, b), lambda l: (l, 0, 0)),
        out_shape=jax.ShapeDtypeStruct((sl, E, b), jnp.float32),
    )(x2, w)


def kernel(sent1, sent2, table, W1, W2):
    b, sl = sent1.shape
    n_rows = 2 * b * sl
    n = table.shape[0]
    packed = _tc_pack(table.T, n)
    src = packed.reshape(-1, E)

    s = BP.bit_length() - 1

    def prep(sent):
        idx = sent.T.reshape(-1).astype(jnp.int32)
        jdx = (
            ((idx >> (s + 1)) << (s + 1))
            + 2 * (idx & (BP - 1))
            + ((idx >> s) & 1)
        )
        jdx = jdx.reshape(sl, 2, b // 2).transpose(0, 2, 1).reshape(-1)
        return jdx.reshape(NW, -1, IDXW)

    e1 = _sc_gather(src, prep(sent1), n_rows // 2)
    e2 = _sc_gather(src, prep(sent2), n_rows // 2)

    o1 = _tc_matmul(e1.reshape(sl, b // 2, 2 * E), W1, sl, b, 0)
    o2 = _tc_matmul(e2.reshape(sl, b // 2, 2 * E), W2, sl, b, 0)

    s1 = o1.transpose(2, 0, 1)
    s2 = o2.transpose(2, 0, 1)
    return (s1, s2)
